# baseline (speedup 1.0000x reference)
import functools

import jax
import jax.numpy as jnp
from jax import lax
from jax.experimental import pallas as pl
from jax.experimental.pallas import tpu as pltpu
from jax.experimental.pallas import tpu_sc as plsc

BATCH = 16384
EMBED_DIM = 64
NUM_CLASSES = 2

_info = plsc.get_sparse_core_info()
_NC, _NS, _L = _info.num_cores, _info.num_subcores, _info.num_lanes
_NW = _NC * _NS
_BPW = BATCH // _NW
_GROUPS = _BPW // _L
_NCH = EMBED_DIM // _L

_mesh = plsc.VectorSubcoreMesh(core_axis_name="c", subcore_axis_name="s")


@functools.partial(
    pl.kernel,
    mesh=_mesh,
    compiler_params=pltpu.CompilerParams(needs_layout_passes=False,
                                         use_tc_tiling_on_sc=False),
    out_type=jax.ShapeDtypeStruct((BATCH * NUM_CLASSES,), jnp.float32),
    scratch_types=[
        pltpu.VMEM((_BPW,), jnp.int32),
        pltpu.VMEM((_BPW, EMBED_DIM), jnp.float32),
        pltpu.VMEM((EMBED_DIM * NUM_CLASSES,), jnp.float32),
        pltpu.VMEM((16,), jnp.float32),
        pltpu.VMEM((16 * 17,), jnp.float32),
        pltpu.VMEM((_BPW * NUM_CLASSES,), jnp.float32),
        pltpu.SemaphoreType.DMA,
    ],
)
def _embed_fwd(idx_hbm, table_hbm, wt_hbm, b_hbm, out_hbm,
               idx_v, rows_v, w_v, b_v, tpad_v, out_v, sem):
    wid = lax.axis_index("s") * _NC + lax.axis_index("c")
    base = wid * _BPW

    pltpu.sync_copy(idx_hbm.at[pl.ds(base, _BPW)], idx_v)
    pltpu.sync_copy(wt_hbm, w_v)
    pltpu.sync_copy(b_hbm, b_v)
    pltpu.async_copy(table_hbm.at[idx_v], rows_v, sem).wait()

    lane = lax.iota(jnp.int32, 16)
    bvec = b_v[...]
    db = bvec[0] - bvec[1]
    wd = [w_v[pl.ds(c * 16, 16)] - w_v[pl.ds(EMBED_DIM + c * 16, 16)]
          for c in range(_NCH)]
    zeros = jnp.zeros((16,), jnp.float32)
    ones = zeros + 1.0

    def group(g, carry):
        rbase = g * 16
        for l in range(16):
            pv = rows_v[rbase + l, pl.ds(0, 16)] * wd[0]
            for c in range(1, _NCH):
                pv = pv + rows_v[rbase + l, pl.ds(c * 16, 16)] * wd[c]
            plsc.store_scatter(tpad_v, [lane * 17 + l], pv)
        delta = tpad_v[pl.ds(0, 16)]
        for j in range(1, 16):
            delta = delta + tpad_v[pl.ds(j * 17, 16)]
        delta = delta + db
        e = jnp.exp(-delta)
        p0 = ones / (ones + e)
        p1 = 1.0 - p0
        ob = g * 32 + lane * 2
        plsc.store_scatter(out_v, [ob], p0)
        plsc.store_scatter(out_v, [ob + 1], p1)
        return carry

    lax.fori_loop(0, _GROUPS, group, 0)
    pltpu.sync_copy(out_v, out_hbm.at[pl.ds(base * NUM_CLASSES,
                                            _BPW * NUM_CLASSES)])


def kernel(inputs, table, W, b):
    idx = inputs.astype(jnp.int32)
    b16 = jnp.pad(b.astype(jnp.float32), (0, 16 - NUM_CLASSES))
    out = _embed_fwd(idx, table, W.T.reshape(-1), b16)
    return out.reshape(BATCH, NUM_CLASSES)


# speedup vs baseline: 3.7876x; 3.7876x over previous
import functools

import jax
import jax.numpy as jnp
from jax import lax
from jax.experimental import pallas as pl
from jax.experimental.pallas import tpu as pltpu
from jax.experimental.pallas import tpu_sc as plsc

BATCH = 16384
EMBED_DIM = 64
NUM_CLASSES = 2
NUM_ROWS = 1000000

_info = plsc.get_sparse_core_info()
_NC, _NS, _L = _info.num_cores, _info.num_subcores, _info.num_lanes
_NW = _NC * _NS
_BPW = BATCH // _NW
_CHUNKS = _BPW // _L

_BW = 8192
_NBLK = (NUM_ROWS + _BW - 1) // _BW

_mesh = plsc.VectorSubcoreMesh(core_axis_name="c", subcore_axis_name="s")


def _dot_block(w_ref, tt_ref, o_ref):
    wd = w_ref[:, 0:1] - w_ref[:, 1:2]
    o_ref[...] = jnp.sum(tt_ref[...] * wd, axis=0)


_table_dot = pl.pallas_call(
    _dot_block,
    grid=(_NBLK,),
    in_specs=[
        pl.BlockSpec((EMBED_DIM, NUM_CLASSES), lambda i: (0, 0)),
        pl.BlockSpec((EMBED_DIM, _BW), lambda i: (0, i)),
    ],
    out_specs=pl.BlockSpec((_BW,), lambda i: (i,)),
    out_shape=jax.ShapeDtypeStruct((NUM_ROWS,), jnp.float32),
)


@functools.partial(
    pl.kernel,
    mesh=_mesh,
    compiler_params=pltpu.CompilerParams(needs_layout_passes=False,
                                         use_tc_tiling_on_sc=False),
    out_type=jax.ShapeDtypeStruct((BATCH * NUM_CLASSES,), jnp.float32),
    scratch_types=[
        pltpu.VMEM((_BPW,), jnp.int32),
        pltpu.VMEM((_BPW,), jnp.float32),
        pltpu.VMEM((16,), jnp.float32),
        pltpu.VMEM((_BPW * NUM_CLASSES,), jnp.float32),
        pltpu.SemaphoreType.DMA,
    ],
)
def _gather_sigmoid(idx_hbm, dbase_hbm, b_hbm, out_hbm,
                    idx_v, d_v, b_v, out_v, sem):
    wid = lax.axis_index("s") * _NC + lax.axis_index("c")
    base = wid * _BPW

    pltpu.sync_copy(idx_hbm.at[pl.ds(base, _BPW)], idx_v)
    pltpu.sync_copy(b_hbm, b_v)
    pltpu.async_copy(dbase_hbm.at[idx_v], d_v, sem).wait()

    lane = lax.iota(jnp.int32, 16)
    bvec = b_v[...]
    db = bvec[0] - bvec[1]
    ones = jnp.zeros((16,), jnp.float32) + 1.0

    def chunk(k, carry):
        d = d_v[pl.ds(k * 16, 16)] + db
        e = jnp.exp(-d)
        p0 = ones / (ones + e)
        p1 = 1.0 - p0
        ob = k * 32 + lane * 2
        plsc.store_scatter(out_v, [ob], p0)
        plsc.store_scatter(out_v, [ob + 1], p1)
        return carry

    lax.fori_loop(0, _CHUNKS, chunk, 0)
    pltpu.sync_copy(out_v, out_hbm.at[pl.ds(base * NUM_CLASSES,
                                            _BPW * NUM_CLASSES)])


def kernel(inputs, table, W, b):
    idx = inputs.astype(jnp.int32)
    b16 = jnp.pad(b.astype(jnp.float32), (0, 16 - NUM_CLASSES))
    dbase = _table_dot(W, table.T)
    out = _gather_sigmoid(idx, dbase, b16)
    return out.reshape(BATCH, NUM_CLASSES)


# speedup vs baseline: 5.4809x; 1.4470x over previous
import functools

import jax
import jax.numpy as jnp
from jax import lax
from jax.experimental import pallas as pl
from jax.experimental.pallas import tpu as pltpu
from jax.experimental.pallas import tpu_sc as plsc

BATCH = 16384
EMBED_DIM = 64
NUM_CLASSES = 2
NUM_ROWS = 1000000

_info = plsc.get_sparse_core_info()
_NC, _NS, _L = _info.num_cores, _info.num_subcores, _info.num_lanes
_NW = _NC * _NS
_BPW = BATCH // _NW
_CHUNKS = _BPW // _L

_BW = 32768
_NBLK = (NUM_ROWS + _BW - 1) // _BW

_mesh = plsc.VectorSubcoreMesh(core_axis_name="c", subcore_axis_name="s")


def _dot_block(w_ref, tt_ref, o_ref):
    wd = w_ref[:, 0:1] - w_ref[:, 1:2]
    o_ref[...] = jnp.sum(tt_ref[...] * wd, axis=0)


_table_dot = pl.pallas_call(
    _dot_block,
    grid=(_NBLK,),
    in_specs=[
        pl.BlockSpec((EMBED_DIM, NUM_CLASSES), lambda i: (0, 0)),
        pl.BlockSpec((EMBED_DIM, _BW), lambda i: (0, i)),
    ],
    out_specs=pl.BlockSpec((_BW,), lambda i: (i,)),
    out_shape=jax.ShapeDtypeStruct((NUM_ROWS,), jnp.float32),
)


@functools.partial(
    pl.kernel,
    mesh=_mesh,
    compiler_params=pltpu.CompilerParams(needs_layout_passes=False,
                                         use_tc_tiling_on_sc=False),
    out_type=jax.ShapeDtypeStruct((BATCH * NUM_CLASSES,), jnp.float32),
    scratch_types=[
        pltpu.VMEM((_BPW,), jnp.int32),
        pltpu.VMEM((_BPW,), jnp.float32),
        pltpu.VMEM((16,), jnp.float32),
        pltpu.VMEM((_BPW * NUM_CLASSES,), jnp.float32),
        pltpu.SemaphoreType.DMA,
    ],
)
def _gather_sigmoid(idx_hbm, dbase_hbm, b_hbm, out_hbm,
                    idx_v, d_v, b_v, out_v, sem):
    wid = lax.axis_index("s") * _NC + lax.axis_index("c")
    base = wid * _BPW

    pltpu.sync_copy(idx_hbm.at[pl.ds(base, _BPW)], idx_v)
    pltpu.sync_copy(b_hbm, b_v)
    pltpu.async_copy(dbase_hbm.at[idx_v], d_v, sem).wait()

    lane = lax.iota(jnp.int32, 16)
    bvec = b_v[...]
    db = bvec[0] - bvec[1]
    ones = jnp.zeros((16,), jnp.float32) + 1.0

    def chunk(k, carry):
        d = d_v[pl.ds(k * 16, 16)] + db
        e = jnp.exp(-d)
        p0 = ones / (ones + e)
        p1 = 1.0 - p0
        ob = k * 32 + lane * 2
        plsc.store_scatter(out_v, [ob], p0)
        plsc.store_scatter(out_v, [ob + 1], p1)
        return carry

    lax.fori_loop(0, _CHUNKS, chunk, 0)
    pltpu.sync_copy(out_v, out_hbm.at[pl.ds(base * NUM_CLASSES,
                                            _BPW * NUM_CLASSES)])


def kernel(inputs, table, W, b):
    idx = inputs.astype(jnp.int32)
    b16 = jnp.pad(b.astype(jnp.float32), (0, 16 - NUM_CLASSES))
    dbase = _table_dot(W, table.T)
    out = _gather_sigmoid(idx, dbase, b16)
    return out.reshape(BATCH, NUM_CLASSES)
